# baseline (speedup 1.0000x reference)
import functools

import jax
import jax.numpy as jnp
from jax import lax
from jax.experimental import pallas as pl
from jax.experimental.pallas import tpu as pltpu
from jax.experimental.pallas import tpu_sc as plsc

N = 10000
E = 160000
D = 256
HID = 256
VAL = 64
SEMB = 8
NC = 40
K = 3
L = 2
TAU = 0.5
LAMBD = 0.1


def _softshrink(x, l):
    return jnp.where(x > l, x - l, jnp.where(x < -l, x + l, jnp.zeros_like(x)))


def _seg_sum(vals, idx, n):
    return jax.ops.segment_sum(vals, idx, num_segments=n)


NPAD = 10240
EPAD = 163840
CH = 128
NW = 32
EPW = EPAD // NW
NCHUNK = EPW // CH
ROWS_PT = NPAD // 16

_MESH = plsc.VectorSubcoreMesh(core_axis_name="c", subcore_axis_name="s")


@functools.partial(jax.jit, static_argnames=("dw", "dt"))
def _sc_gather(tab, idx2d, dw, dt):
    itemsize = 2 if dt == jnp.bfloat16 else 4
    G = max(1, min(4, 40960 // (CH * dw * itemsize // 4)))

    @functools.partial(
        pl.kernel, mesh=_MESH,
        out_type=jax.ShapeDtypeStruct((EPAD, dw), dt),
        compiler_params=pltpu.CompilerParams(use_tc_tiling_on_sc=False),
        scratch_types=[
            pltpu.VMEM((NCHUNK, CH), jnp.int32),
            pltpu.VMEM((G * CH, dw), dt),
            pltpu.VMEM((G * CH, dw), dt),
            pltpu.SemaphoreType.DMA,
            pltpu.SemaphoreType.DMA,
            pltpu.SemaphoreType.DMA,
            pltpu.SemaphoreType.DMA,
        ],
    )
    def gath(tab_hbm, idx_hbm, out_hbm, idx_v, rb0, rb1, g0, g1, s0, s1):
        wid = lax.axis_index("s") * 2 + lax.axis_index("c")
        base = wid * EPW
        pltpu.sync_copy(idx_hbm.at[pl.ds(wid * NCHUNK, NCHUNK)], idx_v)
        G = rb0.shape[0] // CH

        def body(t, carry):
            j0 = 2 * G * t
            ha = [pltpu.async_copy(tab_hbm.at[idx_v.at[j0 + q]],
                                   rb0.at[pl.ds(q * CH, CH)], g0)
                  for q in range(G)]
            hb = [pltpu.async_copy(tab_hbm.at[idx_v.at[j0 + G + q]],
                                   rb1.at[pl.ds(q * CH, CH)], g1)
                  for q in range(G)]
            for h in ha:
                h.wait()
            ca = pltpu.async_copy(rb0, out_hbm.at[pl.ds(base + j0 * CH, G * CH)], s0)
            for h in hb:
                h.wait()
            cb = pltpu.async_copy(
                rb1, out_hbm.at[pl.ds(base + (j0 + G) * CH, G * CH)], s1)
            ca.wait()
            cb.wait()
            return carry

        lax.fori_loop(0, NCHUNK // (2 * G), body, 0)

    return gath(tab, idx2d)


def _zero_acc(vbuf, acc, s, dw):
    def zrow(r, carry):
        def zcol(jj, cc):
            vbuf[r, pl.ds(jj * 16, 16)] = jnp.zeros((16,), jnp.float32)
            return cc
        return lax.fori_loop(0, dw // 16, zcol, carry)

    lax.fori_loop(0, CH, zrow, 0)

    def zcopy(t, carry):
        pltpu.sync_copy(vbuf.at[pl.ds(0, CH)],
                        acc.at[pl.ds(s * ROWS_PT + t * CH, CH)])
        return carry

    lax.fori_loop(0, ROWS_PT // CH, zcopy, 0)


def _drain_acc(vbuf, acc, out_hbm, c, s):
    def drain(t, carry):
        r0 = s * ROWS_PT + t * CH
        pltpu.sync_copy(acc.at[pl.ds(r0, CH)], vbuf.at[pl.ds(0, CH)])
        pltpu.sync_copy(vbuf.at[pl.ds(0, CH)], out_hbm.at[c, pl.ds(r0, CH)])
        return carry

    lax.fori_loop(0, ROWS_PT // CH, drain, 0)


@functools.partial(jax.jit, static_argnames=("dw",))
def _sc_scatter(vals_pad, idx2d, dw):
    @functools.partial(
        pl.kernel, mesh=_MESH,
        out_type=jax.ShapeDtypeStruct((2, NPAD, dw), jnp.float32),
        compiler_params=pltpu.CompilerParams(use_tc_tiling_on_sc=False),
        scratch_types=[
            pltpu.VMEM((NCHUNK, CH), jnp.int32),
            pltpu.VMEM(((4 if dw <= 64 else 2) * CH, dw), jnp.float32),
            pltpu.VMEM_SHARED((NPAD, dw), jnp.float32),
            pltpu.SemaphoreType.DMA,
            pltpu.SemaphoreType.DMA,
        ],
    )
    def scat(vals_hbm, idx_hbm, out_hbm, idx_v, vb0, acc, l0, a0):
        c = lax.axis_index("c")
        s = lax.axis_index("s")
        wid = s * 2 + c
        G = vb0.shape[0] // CH
        _zero_acc(vb0, acc, s, dw)
        pltpu.sync_copy(idx_hbm.at[pl.ds(wid * NCHUNK, NCHUNK)], idx_v)
        plsc.subcore_barrier()

        base = wid * EPW

        def body(t, carry):
            j0 = G * t
            pltpu.async_copy(
                vals_hbm.at[pl.ds(base + j0 * CH, G * CH)], vb0, l0).wait()
            ha = [pltpu.async_copy(vb0.at[pl.ds(q * CH, CH)],
                                   acc.at[idx_v.at[j0 + q]], a0, add=True)
                  for q in range(G)]
            for h in ha:
                h.wait()
            return carry

        lax.fori_loop(0, NCHUNK // G, body, 0)
        plsc.subcore_barrier()
        _drain_acc(vb0, acc, out_hbm, c, s)

    return scat(vals_pad, idx2d)


@functools.partial(jax.jit, static_argnames=("dw",))
def _sc_spmm(tab, src2d, dst2d, dw):
    @functools.partial(
        pl.kernel, mesh=_MESH,
        out_type=jax.ShapeDtypeStruct((2, NPAD, dw), jnp.float32),
        compiler_params=pltpu.CompilerParams(use_tc_tiling_on_sc=False),
        scratch_types=[
            pltpu.VMEM((NCHUNK, CH), jnp.int32),
            pltpu.VMEM((NCHUNK, CH), jnp.int32),
            pltpu.VMEM((2 * CH, dw), jnp.float32),
            pltpu.VMEM_SHARED((NPAD, dw), jnp.float32),
            pltpu.SemaphoreType.DMA,
            pltpu.SemaphoreType.DMA,
        ],
    )
    def spmm(tab_hbm, src_hbm, dst_hbm, out_hbm,
             idx_s, idx_d, vb0, acc, g0, a0):
        c = lax.axis_index("c")
        s = lax.axis_index("s")
        wid = s * 2 + c
        G = vb0.shape[0] // CH
        _zero_acc(vb0, acc, s, dw)
        pltpu.sync_copy(src_hbm.at[pl.ds(wid * NCHUNK, NCHUNK)], idx_s)
        pltpu.sync_copy(dst_hbm.at[pl.ds(wid * NCHUNK, NCHUNK)], idx_d)
        plsc.subcore_barrier()

        def body(t, carry):
            j0 = G * t
            hg = [pltpu.async_copy(tab_hbm.at[idx_s.at[j0 + q]],
                                   vb0.at[pl.ds(q * CH, CH)], g0)
                  for q in range(G)]
            for h in hg:
                h.wait()
            ha = [pltpu.async_copy(vb0.at[pl.ds(q * CH, CH)],
                                   acc.at[idx_d.at[j0 + q]], a0, add=True)
                  for q in range(G)]
            for h in ha:
                h.wait()
            return carry

        lax.fori_loop(0, NCHUNK // G, body, 0)
        plsc.subcore_barrier()
        _drain_acc(vb0, acc, out_hbm, c, s)

    return spmm(tab, src2d, dst2d)


def _pad_e(a):
    pad = [(0, EPAD - E)] + [(0, 0)] * (a.ndim - 1)
    return jnp.pad(a, pad)


def _gather_rows(tab, idx2d):
    return _sc_gather(tab, idx2d, tab.shape[-1], tab.dtype)[:E]


def _scatter_rows(vals, idx2d, n):
    dw = vals.shape[-1]
    vp = _pad_e(vals)
    if dw <= 128:
        ps = _sc_scatter(vp, idx2d, dw)
        return ps[0, :n] + ps[1, :n]
    out = []
    for c0 in range(0, dw, 128):
        ps = _sc_scatter(vp[:, c0:c0 + 128], idx2d, 128)
        out.append(ps[0, :n] + ps[1, :n])
    return jnp.concatenate(out, axis=-1)


BM = 400
BE = 6400


def _mm_body(relu, has_b, has_c, *refs):
    if has_b and has_c:
        a, w, b, c, o = refs
    elif has_b:
        a, w, b, o = refs
        c = None
    elif has_c:
        a, w, c, o = refs
        b = None
    else:
        a, w, o = refs
        b = c = None
    acc = jnp.dot(a[...], w[...], preferred_element_type=jnp.float32)
    if b is not None:
        acc = acc + b[...][None, :]
    if c is not None:
        acc = acc + c[...]
    if relu:
        acc = jnp.maximum(acc, 0.0)
    o[...] = acc


def _tc_mm(A, W, b=None, C=None, relu=False):
    m, k = A.shape
    n = W.shape[1]
    bm = BM if m % BM == 0 else m
    grid = m // bm
    in_specs = [pl.BlockSpec((bm, k), lambda i: (i, 0)),
                pl.BlockSpec((k, n), lambda i: (0, 0))]
    args = [A, W]
    if b is not None:
        in_specs.append(pl.BlockSpec((n,), lambda i: (0,)))
        args.append(b)
    if C is not None:
        in_specs.append(pl.BlockSpec((bm, n), lambda i: (i, 0)))
        args.append(C)
    return pl.pallas_call(
        functools.partial(_mm_body, relu, b is not None, C is not None),
        grid=(grid,),
        in_specs=in_specs,
        out_specs=pl.BlockSpec((bm, n), lambda i: (i, 0)),
        out_shape=jax.ShapeDtypeStruct((m, n), jnp.float32),
    )(*args)


def _s2_alpha_body(ttd, vvs, vj, sgn, semb, am1ws, a2w, sc, ovals, oss):
    base = ttd[...].astype(jnp.float32) + vvs[...].astype(jnp.float32)
    s = sgn[...][:, 0]
    semb2 = jnp.dot(semb[...], am1ws[...], preferred_element_type=jnp.float32)
    sel = (jnp.where(s < -0.5, 1.0, 0.0)[:, None] * semb2[0][None, :]
           + jnp.where(jnp.abs(s) < 0.5, 1.0, 0.0)[:, None] * semb2[1][None, :]
           + jnp.where(s > 0.5, 1.0, 0.0)[:, None] * semb2[2][None, :])
    hdd = jnp.maximum(base + sel, 0.0)
    gamma = sc[0, 0]
    am2b = sc[0, 1]
    alpha = jnp.sum(hdd * a2w[...][None, :], axis=1) + am2b
    alpha = jnp.where(alpha > LAMBD, alpha - LAMBD,
                      jnp.where(alpha < -LAMBD, alpha + LAMBD, 0.0))
    aabs = jnp.abs(alpha)
    weff = jnp.where(s > 0.5, alpha, jnp.where(s < -0.5, -gamma * aabs, 0.0))
    ovals[...] = weff[:, None] * vj[...].astype(jnp.float32)
    oss[...] = jnp.full((8, 128), jnp.sum(aabs), jnp.float32)


def _tc_s2_alpha(ttd, vvs, vj, edge_sign, semb, am1ws, am2w, gamma, am2b):
    grid = E // BE
    sc = jnp.stack([gamma, am2b]).reshape(1, 2)
    sc = jnp.pad(sc, ((0, 0), (0, 126)))
    vals, ss = pl.pallas_call(
        _s2_alpha_body,
        grid=(grid,),
        in_specs=[
            pl.BlockSpec((BE, HID), lambda i: (i, 0)),
            pl.BlockSpec((BE, HID), lambda i: (i, 0)),
            pl.BlockSpec((BE, VAL), lambda i: (i, 0)),
            pl.BlockSpec((BE, 8), lambda i: (i, 0)),
            pl.BlockSpec((3, SEMB), lambda i: (0, 0)),
            pl.BlockSpec((SEMB, HID), lambda i: (0, 0)),
            pl.BlockSpec((HID,), lambda i: (0,)),
            pl.BlockSpec((1, 128), lambda i: (0, 0)),
        ],
        out_specs=[pl.BlockSpec((BE, VAL), lambda i: (i, 0)),
                   pl.BlockSpec((8, 128), lambda i: (i, 0))],
        out_shape=[jax.ShapeDtypeStruct((E, VAL), jnp.float32),
                   jax.ShapeDtypeStruct((grid * 8, 128), jnp.float32)],
    )(ttd, vvs, vj, edge_sign, semb, am1ws, am2w, sc)
    return vals, jnp.sum(ss[::8, 0]) / E


def _em_sign_body(asrc, bdst, w0, w1, w2, g8, sc, osgn):
    hidden = jnp.maximum(asrc[...].astype(jnp.float32)
                         + bdst[...].astype(jnp.float32), 0.0)
    l0 = jnp.sum(hidden * w0[...][None, :], axis=1) + sc[0, 0] + g8[...][:, 0]
    l1 = jnp.sum(hidden * w1[...][None, :], axis=1) + sc[0, 1] + g8[...][:, 1]
    l2 = jnp.sum(hidden * w2[...][None, :], axis=1) + sc[0, 2] + g8[...][:, 2]
    sgn = jnp.where((l2 >= l1) & (l2 >= l0), 1.0,
                    jnp.where(l1 >= l0, 0.0, -1.0))
    osgn[...] = jnp.broadcast_to(sgn[:, None], (sgn.shape[0], 8))


def _tc_em_sign(asrc, bdst, em2W, em2b, g):
    grid = E // BE
    sc = jnp.pad(em2b.reshape(1, 3), ((0, 0), (0, 125)))
    g8 = jnp.pad(g, ((0, 0), (0, 5)))
    espec = pl.BlockSpec((BE, HID), lambda i: (i, 0))
    vspec = pl.BlockSpec((HID,), lambda i: (0,))
    sspec = pl.BlockSpec((BE, 8), lambda i: (i, 0))
    return pl.pallas_call(
        _em_sign_body,
        grid=(grid,),
        in_specs=[espec, espec, vspec, vspec, vspec, sspec,
                  pl.BlockSpec((1, 128), lambda i: (0, 0))],
        out_specs=sspec,
        out_shape=jax.ShapeDtypeStruct((E, 8), jnp.float32),
    )(asrc, bdst, em2W[:, 0], em2W[:, 1], em2W[:, 2], g8, sc)


def _spmm_rows(tab, src2d, dst2d, n):
    dw = tab.shape[-1]
    out = []
    for c0 in range(0, dw, 128):
        ps = _sc_spmm(tab[:, c0:c0 + 128], src2d, dst2d, 128)
        out.append(ps[0, :n] + ps[1, :n])
    return jnp.concatenate(out, axis=-1) if len(out) > 1 else out[0]


def _forward(x, edge_index, y, train_mask, p):
    n = x.shape[0]
    src = edge_index[0]
    dst = edge_index[1]
    src_pad = jnp.pad(src, (0, EPAD - E)).reshape(EPAD // CH, CH)
    dst_pad = jnp.pad(dst, (0, EPAD - E)).reshape(EPAD // CH, CH)
    dst_scat = jnp.pad(dst, (0, EPAD - E),
                       constant_values=NPAD - 1).reshape(EPAD // CH, CH)

    deg = _seg_sum(jnp.ones(E, jnp.float32), dst, n) + 1.0
    dinv = 1.0 / jnp.sqrt(jnp.maximum(deg, 1.0))

    def gcn(h_in, W, b):
        g = dinv[:, None] * _tc_mm(h_in, W)
        agg = _spmm_rows(g, src_pad, dst_scat, n) + g
        return dinv[:, None] * agg + b

    h1 = jax.nn.relu(gcn(x, p['bb1_W'], p['bb1_b']))
    H0 = _tc_mm(x, p['proj_W'], C=gcn(h1, p['bb2_W'], p['bb2_b']), relu=True)

    onehot = jax.nn.one_hot(y, NC, dtype=x.dtype)
    label_feat = onehot * train_mask.astype(x.dtype)[:, None]
    x_in = jnp.concatenate([x, label_feat], axis=-1)

    def gat(h_in, W, a_s, a_d, b):
        h = _tc_mm(h_in, W)
        es = h @ a_s
        ed = h @ a_d
        e_edge = jax.nn.leaky_relu(es[src] + ed[dst], negative_slope=0.2)
        e_self = jax.nn.leaky_relu(es + ed, negative_slope=0.2)
        ee_edge = jnp.exp(e_edge)
        ee_self = jnp.exp(e_self)
        den = _seg_sum(ee_edge, dst, n) + ee_self + 1e-16
        alpha_e = ee_edge / den[dst]
        alpha_s = ee_self / den
        hsrc = _gather_rows(h.astype(jnp.bfloat16), src_pad).astype(jnp.float32)
        out = _scatter_rows(alpha_e[:, None] * hsrc, dst_scat, n) + alpha_s[:, None] * h
        return out + b

    h = jax.nn.relu(gat(x_in, p['gat1_W'], p['gat1_as'], p['gat1_ad'], p['gat1_b']))
    h = gat(h, p['gat2_W'], p['gat2_as'], p['gat2_ad'], p['gat2_b'])

    A = _tc_mm(h, p['em1_W'][:HID])
    B = _tc_mm(h, p['em1_W'][HID:], b=p['em1_b'])
    Asrc = _gather_rows(A.astype(jnp.bfloat16), src_pad)
    Bdst = _gather_rows(B.astype(jnp.bfloat16), dst_pad)

    gkey = jax.random.key(42)
    cls_Wp = jnp.pad(p['cls_W'], ((0, 0), (0, 128 - NC)))
    cls_bp = jnp.pad(p['cls_b'], (0, 128 - NC))
    probs_acc = jnp.zeros((n, NC), jnp.float32)
    sparse_acc = 0.0
    for k in range(K):
        g = jax.random.gumbel(jax.random.fold_in(gkey, k), (E, 3), dtype=x.dtype)
        edge_sign = _tc_em_sign(Asrc, Bdst, p['em2_W'], p['em2_b'], g)
        H = H0
        ss = 0.0
        for l in range(L):
            pref = 'l%d_' % l
            am1W = p[pref + 'am1W']
            Wt2 = _tc_mm(p[pref + 'Wt'], am1W[:VAL])
            Wv2 = _tc_mm(p[pref + 'Wv'], am1W[VAL:2 * VAL])
            TT = _tc_mm(H, Wt2, b=p[pref + 'am1b'])
            VV = _tc_mm(H, Wv2)
            V = _tc_mm(H, p[pref + 'Wv'])
            ttd = _gather_rows(TT.astype(jnp.bfloat16), dst_pad)
            vvs = _gather_rows(VV.astype(jnp.bfloat16), src_pad)
            vj = _gather_rows(V.astype(jnp.bfloat16), src_pad)
            gamma = jax.nn.softplus(p[pref + 'gamma'])
            vals, ssl = _tc_s2_alpha(ttd, vvs, vj, edge_sign,
                                     p[pref + 'semb'], am1W[2 * VAL:],
                                     p[pref + 'am2W'][:, 0], gamma,
                                     p[pref + 'am2b'][0])
            ss = ss + ssl
            signed = _scatter_rows(vals, dst_scat, n)
            S = _tc_mm(signed, p[pref + 'WoutW'], b=p[pref + 'Woutb'], C=H)
            H = _tc_mm(H, p[pref + 'Wself'], C=S, relu=True)
        probs_acc = probs_acc + jax.nn.softmax(
            _tc_mm(H, cls_Wp, b=cls_bp)[:, :NC], axis=-1)
        sparse_acc = sparse_acc + ss / L
    probs_mc = probs_acc / K
    logits_mc = jnp.log(probs_mc + 1e-12)
    sparse_loss = sparse_acc / K
    return logits_mc, sparse_loss


def kernel(x, edge_index, y, train_mask, params):
    return _forward(x, edge_index, y, train_mask, params)
